# baseline (speedup 1.0000x reference)
import functools

import jax
import jax.numpy as jnp
from jax import lax
from jax.experimental import pallas as pl
from jax.experimental.pallas import tpu as pltpu
from jax.experimental.pallas import tpu_sc as plsc

N_NODES = 100000
D_FEAT = 128
N_TASKS = 4
N_GRAPHS = 2048
TD = N_TASKS * D_FEAT

WBLK = 2000


def _w_body(feats_ref, wt_ref, b_ref, w0_ref, w1_ref, w2_ref, w3_ref):
    logits = jax.lax.dot_general(
        wt_ref[...], feats_ref[...], (((1,), (1,)), ((), ())),
        preferred_element_type=jnp.float32)
    w = jax.nn.sigmoid(logits + b_ref[...])
    w0_ref[...] = w[0][None, None, :]
    w1_ref[...] = w[1][None, None, :]
    w2_ref[...] = w[2][None, None, :]
    w3_ref[...] = w[3][None, None, :]


def _tc_weights(feats, W, b):
    nblk = N_NODES // WBLK
    oshape = jax.ShapeDtypeStruct((nblk, 1, WBLK), jnp.float32)
    ospec = pl.BlockSpec((1, 1, WBLK), lambda i: (i, 0, 0))
    return pl.pallas_call(
        _w_body,
        grid=(nblk,),
        in_specs=[
            pl.BlockSpec((WBLK, D_FEAT), lambda i: (i, 0)),
            pl.BlockSpec((N_TASKS, D_FEAT), lambda i: (0, 0)),
            pl.BlockSpec((N_TASKS, 1), lambda i: (0, 0)),
        ],
        out_specs=[ospec, ospec, ospec, ospec],
        out_shape=[oshape, oshape, oshape, oshape],
    )(feats, W, b.reshape(N_TASKS, 1))


NC = 2
NS = 16
NW = NC * NS
RPW = 3128
RPW_LAST = N_NODES - (NW - 1) * RPW
CH = 184
N_CHUNKS = 17
TAIL = 88
GROWS = N_GRAPHS // NS
ZROWS = 16


def _sc_body(feats_hbm, w0_hbm, w1_hbm, w2_hbm, w3_hbm, seg_hbm, out_hbm,
             feats_v, w0_v, w1_v, w2_v, w3_v, seg_v, accbuf, idbuf,
             tmpa, tmpb, zerobuf, acc_sh, bnd_sh, ids_sh):
    w_hbms = (w0_hbm, w1_hbm, w2_hbm, w3_hbm)
    w_vs = (w0_v, w1_v, w2_v, w3_v)
    cid = lax.axis_index("c")
    sid = lax.axis_index("s")
    wid = sid * NC + cid
    base = wid * RPW
    is_last = wid == NW - 1

    zvec = jnp.zeros((16,), jnp.float32)

    for q in range(ZROWS * TD // 16):
        zerobuf[pl.ds(16 * q, 16)] = zvec
    for q in range(GROWS // ZROWS):
        pltpu.sync_copy(
            zerobuf,
            acc_sh.at[pl.ds((sid * GROWS + q * ZROWS) * TD, ZROWS * TD)])
    idbuf[pl.ds(0, 16)] = jnp.full((16,), -1, jnp.int32)
    pltpu.sync_copy(idbuf, ids_sh.at[pl.ds(sid * 32, 16)])
    pltpu.sync_copy(idbuf, ids_sh.at[pl.ds(sid * 32 + 16, 16)])
    plsc.subcore_barrier()

    def stage_meta(n):
        def do():
            pltpu.sync_copy(seg_hbm.at[pl.ds(base, n)], seg_v.at[pl.ds(0, n)])
            for t in range(N_TASKS):
                pltpu.sync_copy(w_hbms[t].at[pl.ds(base, n)],
                                w_vs[t].at[pl.ds(0, n)])
        return do

    lax.cond(is_last, stage_meta(RPW_LAST), stage_meta(RPW))

    @pl.when(is_last)
    def _pad_last():
        s_last = seg_v[pl.ds(RPW_LAST - 16, 16)][15]
        fillv = jnp.full((16,), 1, jnp.int32) * s_last
        zf = jnp.zeros((16,), jnp.float32)
        for q in range(10):
            seg_v[pl.ds(RPW_LAST + 16 * q, 16)] = fillv
            for t in range(N_TASKS):
                w_vs[t][pl.ds(RPW_LAST + 16 * q, 16)] = zf

    first_seg = seg_v[pl.ds(0, 16)][0]

    zeros_acc = tuple(zvec for _ in range(32))

    def spill_acc(acc):
        for q in range(32):
            accbuf[pl.ds(16 * q, 16)] = acc[q]

    def flush(cur, acc):
        spill_acc(acc)

        def to_boundary():
            idbuf[pl.ds(0, 16)] = jnp.full((16,), 1, jnp.int32) * cur
            pltpu.sync_copy(idbuf, ids_sh.at[pl.ds(sid * 32, 16)])
            pltpu.sync_copy(accbuf, bnd_sh.at[pl.ds(sid * 2 * TD, TD)])

        def to_interior():
            pltpu.sync_copy(accbuf, acc_sh.at[pl.ds(cur * TD, TD)])

        lax.cond(cur == first_seg, to_boundary, to_interior)

    def make_row_body(c):
        def row_body(i, carry):
            cur, acc = carry
            r = c * CH + i
            s = seg_v[pl.ds(r, 16)][0]
            changed = s != cur

            @pl.when(changed)
            def _f():
                flush(cur, acc)

            wts = [w_vs[t][pl.ds(r, 16)][0] for t in range(N_TASKS)]
            new_acc = []
            for j in range(8):
                f = feats_v[pl.ds(i * D_FEAT + 16 * j, 16)]
                for t in range(N_TASKS):
                    new_acc.append((t * 8 + j, f * wts[t]))
            new_acc = [v for (_, v) in sorted(new_acc)]
            acc = tuple(jnp.where(changed, nv, av + nv)
                        for (av, nv) in zip(acc, new_acc))
            return (s, acc)
        return row_body

    def chunk_body(c, carry):
        off = base + c * CH
        short = jnp.logical_and(is_last, c == N_CHUNKS - 1)
        pltpu.sync_copy(feats_hbm.at[pl.ds(off * D_FEAT, TAIL * D_FEAT)],
                        feats_v.at[pl.ds(0, TAIL * D_FEAT)])

        @pl.when(jnp.logical_not(short))
        def _rest():
            pltpu.sync_copy(
                feats_hbm.at[pl.ds((off + TAIL) * D_FEAT,
                                   (CH - TAIL) * D_FEAT)],
                feats_v.at[pl.ds(TAIL * D_FEAT, (CH - TAIL) * D_FEAT)])

        @pl.when(short)
        def _zero_rest():
            zf = jnp.zeros((16,), jnp.float32)
            for q in range((CH - TAIL) * D_FEAT // 16):
                feats_v[pl.ds(TAIL * D_FEAT + 16 * q, 16)] = zf

        return lax.fori_loop(0, CH, make_row_body(c), carry, unroll=16)

    carry0 = (first_seg, zeros_acc)
    cur, acc = lax.fori_loop(0, N_CHUNKS, chunk_body, carry0, unroll=False)

    spill_acc(acc)
    idbuf[pl.ds(0, 16)] = jnp.full((16,), 1, jnp.int32) * cur
    pltpu.sync_copy(idbuf, ids_sh.at[pl.ds(sid * 32 + 16, 16)])
    pltpu.sync_copy(accbuf, bnd_sh.at[pl.ds((sid * 2 + 1) * TD, TD)])

    plsc.subcore_barrier()

    @pl.when(sid == 0)
    def _combine():
        def slot_body(s, carry):
            pltpu.sync_copy(ids_sh.at[pl.ds(s * 16, 16)], idbuf)
            idq = idbuf[pl.ds(0, 16)][0]

            @pl.when(idq >= 0)
            def _add():
                pltpu.sync_copy(bnd_sh.at[pl.ds(s * TD, TD)], tmpb)
                pltpu.sync_copy(acc_sh.at[pl.ds(idq * TD, TD)], tmpa)
                for q in range(32):
                    tmpa[pl.ds(16 * q, 16)] = (
                        tmpa[pl.ds(16 * q, 16)] + tmpb[pl.ds(16 * q, 16)])
                pltpu.sync_copy(tmpa, acc_sh.at[pl.ds(idq * TD, TD)])

            return carry

        lax.fori_loop(0, 2 * NS, slot_body, jnp.int32(0), unroll=False)

    plsc.subcore_barrier()
    pltpu.sync_copy(acc_sh.at[pl.ds(sid * GROWS * TD, GROWS * TD)],
                    out_hbm.at[cid, pl.ds(sid * GROWS * TD, GROWS * TD)])


@functools.partial(
    pl.kernel,
    out_type=jax.ShapeDtypeStruct((NC, N_GRAPHS * TD), jnp.float32),
    mesh=plsc.VectorSubcoreMesh(core_axis_name="c", subcore_axis_name="s"),
    scratch_types=[
        pltpu.VMEM((CH * D_FEAT,), jnp.float32),
        pltpu.VMEM((RPW + 160,), jnp.float32),
        pltpu.VMEM((RPW + 160,), jnp.float32),
        pltpu.VMEM((RPW + 160,), jnp.float32),
        pltpu.VMEM((RPW + 160,), jnp.float32),
        pltpu.VMEM((RPW + 160,), jnp.int32),
        pltpu.VMEM((TD,), jnp.float32),
        pltpu.VMEM((16,), jnp.int32),
        pltpu.VMEM((TD,), jnp.float32),
        pltpu.VMEM((TD,), jnp.float32),
        pltpu.VMEM((ZROWS * TD,), jnp.float32),
        pltpu.VMEM_SHARED((N_GRAPHS * TD,), jnp.float32),
        pltpu.VMEM_SHARED((NS * 2 * TD,), jnp.float32),
        pltpu.VMEM_SHARED((NS * 32,), jnp.int32),
    ],
)
def _sc_segsum(feats_hbm, w0_hbm, w1_hbm, w2_hbm, w3_hbm, seg_hbm, out_hbm,
               feats_v, w0_v, w1_v, w2_v, w3_v, seg_v, accbuf, idbuf,
               tmpa, tmpb, zerobuf, acc_sh, bnd_sh, ids_sh):
    _sc_body(feats_hbm, w0_hbm, w1_hbm, w2_hbm, w3_hbm, seg_hbm, out_hbm,
             feats_v, w0_v, w1_v, w2_v, w3_v, seg_v, accbuf, idbuf,
             tmpa, tmpb, zerobuf, acc_sh, bnd_sh, ids_sh)


def kernel(feats, segment_ids, W, b):
    seg32 = segment_ids.astype(jnp.int32)
    ws = [w.reshape(N_NODES) for w in _tc_weights(feats, W, b)]
    partial = _sc_segsum(feats.reshape(-1), ws[0], ws[1], ws[2], ws[3],
                         seg32)
    acc = partial[0] + partial[1]
    readout = acc.reshape(N_GRAPHS, N_TASKS, D_FEAT).transpose(1, 0, 2)
    atoms = jnp.stack(ws).reshape(N_TASKS, N_NODES, 1)
    return (readout, atoms)


# speedup vs baseline: 1.1608x; 1.1608x over previous
import functools

import jax
import jax.numpy as jnp
from jax import lax
from jax.experimental import pallas as pl
from jax.experimental.pallas import tpu as pltpu
from jax.experimental.pallas import tpu_sc as plsc

N_NODES = 100000
D_FEAT = 128
N_TASKS = 4
N_GRAPHS = 2048
TD = N_TASKS * D_FEAT

WBLK = 2000


def _w_body(feats_ref, wt_ref, b_ref, w0_ref, w1_ref, w2_ref, w3_ref):
    logits = jax.lax.dot_general(
        wt_ref[...], feats_ref[...], (((1,), (1,)), ((), ())),
        preferred_element_type=jnp.float32)
    w = jax.nn.sigmoid(logits + b_ref[...])
    w0_ref[...] = w[0][None, None, :]
    w1_ref[...] = w[1][None, None, :]
    w2_ref[...] = w[2][None, None, :]
    w3_ref[...] = w[3][None, None, :]


def _tc_weights(feats, W, b):
    nblk = N_NODES // WBLK
    oshape = jax.ShapeDtypeStruct((nblk, 1, WBLK), jnp.float32)
    ospec = pl.BlockSpec((1, 1, WBLK), lambda i: (i, 0, 0))
    return pl.pallas_call(
        _w_body,
        grid=(nblk,),
        in_specs=[
            pl.BlockSpec((WBLK, D_FEAT), lambda i: (i, 0)),
            pl.BlockSpec((N_TASKS, D_FEAT), lambda i: (0, 0)),
            pl.BlockSpec((N_TASKS, 1), lambda i: (0, 0)),
        ],
        out_specs=[ospec, ospec, ospec, ospec],
        out_shape=[oshape, oshape, oshape, oshape],
    )(feats, W, b.reshape(N_TASKS, 1))


NC = 2
NS = 16
NW = NC * NS
RPW = 3128
RPW_LAST = N_NODES - (NW - 1) * RPW
CH = 136
N_CHUNKS = 23
TAIL = 40
GROWS = N_GRAPHS // NS
ZROWS = 8


def _sc_body(feats_hbm, w0_hbm, w1_hbm, w2_hbm, w3_hbm, seg_hbm, out_hbm,
             feats_v, sem0, sem1, w0_v, w1_v, w2_v, w3_v, seg_v, accbuf,
             idbuf, tmpa, tmpb, zerobuf, acc_sh, bnd_sh, ids_sh):
    w_hbms = (w0_hbm, w1_hbm, w2_hbm, w3_hbm)
    w_vs = (w0_v, w1_v, w2_v, w3_v)
    cid = lax.axis_index("c")
    sid = lax.axis_index("s")
    wid = sid * NC + cid
    base = wid * RPW
    is_last = wid == NW - 1

    zvec = jnp.zeros((16,), jnp.float32)

    for q in range(ZROWS * TD // 16):
        zerobuf[pl.ds(16 * q, 16)] = zvec
    for q in range(GROWS // ZROWS):
        pltpu.sync_copy(
            zerobuf,
            acc_sh.at[pl.ds((sid * GROWS + q * ZROWS) * TD, ZROWS * TD)])
    idbuf[pl.ds(0, 16)] = jnp.full((16,), -1, jnp.int32)
    pltpu.sync_copy(idbuf, ids_sh.at[pl.ds(sid * 32, 16)])
    pltpu.sync_copy(idbuf, ids_sh.at[pl.ds(sid * 32 + 16, 16)])
    plsc.subcore_barrier()

    def stage_meta(n):
        def do():
            pltpu.sync_copy(seg_hbm.at[pl.ds(base, n)], seg_v.at[pl.ds(0, n)])
            for t in range(N_TASKS):
                pltpu.sync_copy(w_hbms[t].at[pl.ds(base, n)],
                                w_vs[t].at[pl.ds(0, n)])
        return do

    lax.cond(is_last, stage_meta(RPW_LAST), stage_meta(RPW))

    @pl.when(is_last)
    def _pad_last():
        s_last = seg_v[pl.ds(RPW_LAST - 16, 16)][15]
        fillv = jnp.full((16,), 1, jnp.int32) * s_last
        zf = jnp.zeros((16,), jnp.float32)
        for q in range(10):
            seg_v[pl.ds(RPW_LAST + 16 * q, 16)] = fillv
            for t in range(N_TASKS):
                w_vs[t][pl.ds(RPW_LAST + 16 * q, 16)] = zf

    first_seg = seg_v[pl.ds(0, 16)][0]

    zeros_acc = tuple(zvec for _ in range(32))

    def spill_acc(acc):
        for q in range(32):
            accbuf[pl.ds(16 * q, 16)] = acc[q]

    def flush(cur, acc):
        spill_acc(acc)

        def to_boundary():
            idbuf[pl.ds(0, 16)] = jnp.full((16,), 1, jnp.int32) * cur
            pltpu.sync_copy(idbuf, ids_sh.at[pl.ds(sid * 32, 16)])
            pltpu.sync_copy(accbuf, bnd_sh.at[pl.ds(sid * 2 * TD, TD)])

        def to_interior():
            pltpu.sync_copy(accbuf, acc_sh.at[pl.ds(cur * TD, TD)])

        lax.cond(cur == first_seg, to_boundary, to_interior)

    def make_row_body(c):
        boff = (c % 2) * CH * D_FEAT

        def row_body(i, carry):
            cur, acc = carry
            r = c * CH + i
            s = seg_v[pl.ds(r, 16)][0]
            changed = s != cur

            @pl.when(changed)
            def _f():
                flush(cur, acc)

            wts = [w_vs[t][pl.ds(r, 16)][0] for t in range(N_TASKS)]
            new_acc = []
            for j in range(8):
                f = feats_v[pl.ds(boff + i * D_FEAT + 16 * j, 16)]
                for t in range(N_TASKS):
                    new_acc.append((t * 8 + j, f * wts[t]))
            new_acc = [v for (_, v) in sorted(new_acc)]
            acc = tuple(jnp.where(changed, nv, av + nv)
                        for (av, nv) in zip(acc, new_acc))
            return (s, acc)
        return row_body

    def _parts(k):
        off = base + k * CH
        boff = (k % 2) * CH * D_FEAT
        shortk = jnp.logical_and(is_last, k == N_CHUNKS - 1)
        head = (feats_hbm.at[pl.ds(off * D_FEAT, TAIL * D_FEAT)],
                feats_v.at[pl.ds(boff, TAIL * D_FEAT)])
        rest = (feats_hbm.at[pl.ds((off + TAIL) * D_FEAT,
                                   (CH - TAIL) * D_FEAT)],
                feats_v.at[pl.ds(boff + TAIL * D_FEAT,
                                 (CH - TAIL) * D_FEAT)])
        return head, rest, shortk

    def issue_chunk(k, sem):
        head, rest, shortk = _parts(k)
        pltpu.async_copy(head[0], head[1], sem)

        @pl.when(jnp.logical_not(shortk))
        def _r():
            pltpu.async_copy(rest[0], rest[1], sem)

    def wait_chunk(k, sem):
        head, rest, shortk = _parts(k)
        pltpu.make_async_copy(head[0], head[1], sem).wait()

        @pl.when(jnp.logical_not(shortk))
        def _r():
            pltpu.make_async_copy(rest[0], rest[1], sem).wait()

    def by_parity(k, fn):
        @pl.when(k % 2 == 0)
        def _e():
            fn(k, sem0)

        @pl.when(k % 2 == 1)
        def _o():
            fn(k, sem1)

    def chunk_body(c, carry):
        boff = (c % 2) * CH * D_FEAT
        short = jnp.logical_and(is_last, c == N_CHUNKS - 1)
        by_parity(c, wait_chunk)

        @pl.when(c + 1 < N_CHUNKS)
        def _prefetch():
            by_parity(c + 1, issue_chunk)

        @pl.when(short)
        def _zero_rest():
            zf = jnp.zeros((16,), jnp.float32)
            for q in range((CH - TAIL) * D_FEAT // 16):
                feats_v[pl.ds(boff + TAIL * D_FEAT + 16 * q, 16)] = zf

        return lax.fori_loop(0, CH, make_row_body(c), carry, unroll=8)

    issue_chunk(jnp.int32(0), sem0)
    carry0 = (first_seg, zeros_acc)
    cur, acc = lax.fori_loop(0, N_CHUNKS, chunk_body, carry0, unroll=False)

    spill_acc(acc)
    idbuf[pl.ds(0, 16)] = jnp.full((16,), 1, jnp.int32) * cur
    pltpu.sync_copy(idbuf, ids_sh.at[pl.ds(sid * 32 + 16, 16)])
    pltpu.sync_copy(accbuf, bnd_sh.at[pl.ds((sid * 2 + 1) * TD, TD)])

    plsc.subcore_barrier()

    @pl.when(sid == 0)
    def _combine():
        def slot_body(s, carry):
            pltpu.sync_copy(ids_sh.at[pl.ds(s * 16, 16)], idbuf)
            idq = idbuf[pl.ds(0, 16)][0]

            @pl.when(idq >= 0)
            def _add():
                pltpu.sync_copy(bnd_sh.at[pl.ds(s * TD, TD)], tmpb)
                pltpu.sync_copy(acc_sh.at[pl.ds(idq * TD, TD)], tmpa)
                for q in range(32):
                    tmpa[pl.ds(16 * q, 16)] = (
                        tmpa[pl.ds(16 * q, 16)] + tmpb[pl.ds(16 * q, 16)])
                pltpu.sync_copy(tmpa, acc_sh.at[pl.ds(idq * TD, TD)])

            return carry

        lax.fori_loop(0, 2 * NS, slot_body, jnp.int32(0), unroll=False)

    plsc.subcore_barrier()
    pltpu.sync_copy(acc_sh.at[pl.ds(sid * GROWS * TD, GROWS * TD)],
                    out_hbm.at[cid, pl.ds(sid * GROWS * TD, GROWS * TD)])


@functools.partial(
    pl.kernel,
    out_type=jax.ShapeDtypeStruct((NC, N_GRAPHS * TD), jnp.float32),
    mesh=plsc.VectorSubcoreMesh(core_axis_name="c", subcore_axis_name="s"),
    scratch_types=[
        pltpu.VMEM((2 * CH * D_FEAT,), jnp.float32),
        pltpu.SemaphoreType.DMA,
        pltpu.SemaphoreType.DMA,
        pltpu.VMEM((RPW + 160,), jnp.float32),
        pltpu.VMEM((RPW + 160,), jnp.float32),
        pltpu.VMEM((RPW + 160,), jnp.float32),
        pltpu.VMEM((RPW + 160,), jnp.float32),
        pltpu.VMEM((RPW + 160,), jnp.int32),
        pltpu.VMEM((TD,), jnp.float32),
        pltpu.VMEM((16,), jnp.int32),
        pltpu.VMEM((TD,), jnp.float32),
        pltpu.VMEM((TD,), jnp.float32),
        pltpu.VMEM((ZROWS * TD,), jnp.float32),
        pltpu.VMEM_SHARED((N_GRAPHS * TD,), jnp.float32),
        pltpu.VMEM_SHARED((NS * 2 * TD,), jnp.float32),
        pltpu.VMEM_SHARED((NS * 32,), jnp.int32),
    ],
)
def _sc_segsum(feats_hbm, w0_hbm, w1_hbm, w2_hbm, w3_hbm, seg_hbm, out_hbm,
               feats_v, sem0, sem1, w0_v, w1_v, w2_v, w3_v, seg_v, accbuf,
               idbuf, tmpa, tmpb, zerobuf, acc_sh, bnd_sh, ids_sh):
    _sc_body(feats_hbm, w0_hbm, w1_hbm, w2_hbm, w3_hbm, seg_hbm, out_hbm,
             feats_v, sem0, sem1, w0_v, w1_v, w2_v, w3_v, seg_v, accbuf,
             idbuf, tmpa, tmpb, zerobuf, acc_sh, bnd_sh, ids_sh)


def kernel(feats, segment_ids, W, b):
    seg32 = segment_ids.astype(jnp.int32)
    ws = [w.reshape(N_NODES) for w in _tc_weights(feats, W, b)]
    partial = _sc_segsum(feats.reshape(-1), ws[0], ws[1], ws[2], ws[3],
                         seg32)
    acc = partial[0] + partial[1]
    readout = acc.reshape(N_GRAPHS, N_TASKS, D_FEAT).transpose(1, 0, 2)
    atoms = jnp.stack(ws).reshape(N_TASKS, N_NODES, 1)
    return (readout, atoms)
